# baseline (speedup 1.0000x reference)
import functools

import jax
import jax.numpy as jnp
from jax import lax
from jax.experimental import pallas as pl
from jax.experimental.pallas import tpu as pltpu
from jax.experimental.pallas import tpu_sc as plsc

_N_OUT = 4096
_N_IN = 4096
_M = _N_OUT * _N_IN

_NC = 2
_NS = 16
_NW = _NC * _NS

_CPW = 16
_CHW = _M // (_NW * _CPW)
_ROWS = _CHW // _N_IN

_BATCH = 512
_LANES = 16
_NBUF = 3

_mesh = plsc.VectorSubcoreMesh(
    core_axis_name="c", subcore_axis_name="s", num_cores=_NC, num_subcores=_NS
)


@functools.lru_cache(maxsize=None)
def _make_fused(n):
  steps, ln = 0, n
  while ln > 0:
    q = ln >> 2
    ln = max(q, ln - 3 * q - 1)
    steps += 1

  @functools.partial(
      pl.kernel,
      out_type=jax.ShapeDtypeStruct((_N_OUT, _N_IN), jnp.float32),
      mesh=_mesh,
      compiler_params=pltpu.CompilerParams(needs_layout_passes=False),
      scratch_types=[
          pltpu.VMEM((_ROWS, _N_IN), jnp.float32),
          pltpu.VMEM((_ROWS, _N_IN), jnp.float32),
          pltpu.VMEM((_ROWS, _N_IN), jnp.float32),
          pltpu.VMEM((_BATCH + _LANES,), jnp.int32),
          pltpu.VMEM((_BATCH + _LANES,), jnp.int32),
          pltpu.VMEM((_BATCH,), jnp.float32),
          pltpu.VMEM((_BATCH,), jnp.float32),
          pltpu.VMEM((3, _LANES), jnp.int32),
          pltpu.VMEM((3, _LANES), jnp.int32),
          pltpu.SemaphoreType.DMA,
          pltpu.SemaphoreType.DMA,
          pltpu.SemaphoreType.DMA,
          pltpu.SemaphoreType.DMA,
          pltpu.SemaphoreType.DMA,
          pltpu.SemaphoreType.DMA,
          pltpu.SemaphoreType.DMA,
          pltpu.SemaphoreType.DMA,
          pltpu.SemaphoreType.DMA,
          pltpu.SemaphoreType.DMA,
      ],
  )
  def _fused_knl(w_hbm, idx_hbm, val_hbm, out_hbm,
                 buf0, buf1, buf2, sidx0, sidx1, sval0, sval1, ps_v, pe_v,
                 isem0, isem1, isem2, osem0, osem1, osem2, ssem0, ssem1,
                 psem_s, psem_e):
    wid = lax.axis_index("s") * _NC + lax.axis_index("c")
    row0 = wid * (_CPW * _ROWS)
    bufs = (buf0, buf1, buf2)
    isems = (isem0, isem1, isem2)
    osems = (osem0, osem1, osem2)
    sidx = (sidx0, sidx1)
    sval = (sval0, sval1)
    ssems = (ssem0, ssem1)
    lane = lax.iota(jnp.int32, _LANES)

    def in_dma(c):
      r = row0 + c * _ROWS
      return pltpu.make_async_copy(
          w_hbm.at[pl.ds(r, _ROWS)], bufs[c % _NBUF], isems[c % _NBUF])

    def out_dma(c):
      r = row0 + c * _ROWS
      return pltpu.make_async_copy(
          bufs[c % _NBUF], out_hbm.at[pl.ds(r, _ROWS)], osems[c % _NBUF])

    in_dma(0).start()
    in_dma(1).start()

    cut_s = (wid * _CPW + lane) * _CHW
    cut_e = cut_s + _CHW
    lo_s = jnp.zeros((_LANES,), jnp.int32)
    lo_e = jnp.zeros((_LANES,), jnp.int32)
    hi_s = jnp.full((_LANES,), n, jnp.int32)
    hi_e = jnp.full((_LANES,), n, jnp.int32)
    for _ in range(steps):
      q_s = (hi_s - lo_s) >> 2
      q_e = (hi_e - lo_e) >> 2
      m_s = (lo_s + q_s, lo_s + 2 * q_s, lo_s + 3 * q_s)
      m_e = (lo_e + q_e, lo_e + 2 * q_e, lo_e + 3 * q_e)
      cps = [
          pltpu.make_async_copy(idx_hbm.at[m_s[j]], ps_v.at[j], psem_s)
          for j in range(3)
      ] + [
          pltpu.make_async_copy(idx_hbm.at[m_e[j]], pe_v.at[j], psem_e)
          for j in range(3)
      ]
      for cp in cps:
        cp.start()
      for cp in cps:
        cp.wait()
      ne_s = lo_s < hi_s
      ne_e = lo_e < hi_e
      b_s = [(ps_v[j, :] < cut_s) & ne_s for j in range(3)]
      b_e = [(pe_v[j, :] < cut_e) & ne_e for j in range(3)]
      lo_s = jnp.where(
          b_s[2], m_s[2] + 1,
          jnp.where(b_s[1], m_s[1] + 1, jnp.where(b_s[0], m_s[0] + 1, lo_s)))
      hi_s = jnp.where(
          b_s[0], jnp.where(b_s[1], jnp.where(b_s[2], hi_s, m_s[2]), m_s[1]),
          m_s[0])
      lo_e = jnp.where(
          b_e[2], m_e[2] + 1,
          jnp.where(b_e[1], m_e[1] + 1, jnp.where(b_e[0], m_e[0] + 1, lo_e)))
      hi_e = jnp.where(
          b_e[0], jnp.where(b_e[1], jnp.where(b_e[2], hi_e, m_e[2]), m_e[1]),
          m_e[0])

    aa = [pl.multiple_of(lo_s[c] & ~jnp.int32(7), 8) for c in range(_CPW)]
    ss = [lo_s[c] for c in range(_CPW)]
    ee = [lo_e[c] for c in range(_CPW)]

    def stage_dmas(c):
      return (
          pltpu.make_async_copy(
              idx_hbm.at[pl.ds(aa[c], _BATCH + _LANES)], sidx[c % 2],
              ssems[c % 2]),
          pltpu.make_async_copy(
              val_hbm.at[pl.ds(aa[c], _BATCH)], sval[c % 2], ssems[c % 2]),
      )

    for d in stage_dmas(0):
      d.start()

    for c in range(_CPW):
      if c + 2 < _CPW:
        if c >= 1:
          out_dma(c - 1).wait()
        in_dma(c + 2).start()
      if c + 1 < _CPW:
        for d in stage_dmas(c + 1):
          d.start()
      in_dma(c).wait()
      for d in stage_dmas(c):
        d.wait()

      buf = bufs[c % _NBUF]
      idx_v = sidx[c % 2]
      val_v = sval[c % 2]
      chunk_row = row0 + c * _ROWS
      s, e, a = ss[c], ee[c], aa[c]
      nb = (e - a + jnp.int32(_BATCH - 1)) // _BATCH

      def batch_body(b):
        off0 = pl.multiple_of(a + b * _BATCH, 8)

        @pl.when(b > 0)
        def _restage():
          pltpu.sync_copy(idx_hbm.at[pl.ds(off0, _BATCH + _LANES)], idx_v)
          pltpu.sync_copy(val_hbm.at[pl.ds(off0, _BATCH)], val_v)

        for g in range(_BATCH // _LANES):
          p = off0 + g * _LANES + lane
          m = (p >= s) & (p < e)
          gi = idx_v[pl.ds(g * _LANES, _LANES)]
          gv = val_v[pl.ds(g * _LANES, _LANES)]
          gnext = idx_v[pl.ds(g * _LANES + 1, _LANES)]
          m = m & (gi != gnext)
          loc_r = ((gi >> 12) - chunk_row) & jnp.int32(_ROWS - 1)
          loc_c = gi & jnp.int32(_N_IN - 1)
          plsc.store_scatter(buf, [loc_r, loc_c], gv, mask=m)

      lax.fori_loop(0, nb, lambda b, _: (batch_body(b), 0)[1], 0)
      out_dma(c).start()
    out_dma(_CPW - 3).wait()
    out_dma(_CPW - 2).wait()
    out_dma(_CPW - 1).wait()

  return _fused_knl


def kernel(quantized_weight, flat_idx, sparse_values):
  n = flat_idx.shape[0]
  pad = _BATCH + 2 * _LANES + 8
  idx_p = jnp.concatenate([flat_idx, jnp.full((pad,), -1, flat_idx.dtype)])
  val_p = jnp.concatenate([sparse_values, jnp.zeros((pad,), sparse_values.dtype)])
  return _make_fused(n)(quantized_weight, idx_p, val_p)
